# baseline (speedup 1.0000x reference)
import functools

import jax
import jax.numpy as jnp
from jax import lax
from jax.experimental import pallas as pl
from jax.experimental.pallas import tpu as pltpu
from jax.experimental.pallas import tpu_sc as plsc

NUM_USERS = 25000
NUM_ITEMS = 25000
N = NUM_USERS + NUM_ITEMS
E = 800000
D = 64
G = 3
REG_LAMBDA = 1e-4
B = 4096

NSUB = 16
NCORE = 2
WIN = 512
PER_SUB = 50176
NWIN = PER_SUB // WIN
E_PAD = NSUB * PER_SUB
N_ACC = 50176
ROWS_PER_SUB = N_ACC // NSUB
ZCHUNK = 448


def _sc_mesh():
    return plsc.VectorSubcoreMesh(core_axis_name="c", subcore_axis_name="s")


def _make_sc_pass(table_rows, table_w, table_dtype, acc_dtype, core_off,
                  idx_mult, use_pat, win, phases=1):
    need_sbuf = table_dtype == jnp.bfloat16 and acc_dtype == jnp.float32
    acc_w = table_w
    nwin = PER_SUB // win
    nsc = win // 128
    zchunk = 448 if win >= 448 else (224 if win >= 224 else 112)
    nz = ROWS_PER_SUB // zchunk

    def dbl(t):
        return [t, t]

    scratch = []
    if use_pat:
        scratch += dbl(pltpu.VMEM((nsc, 128), jnp.int32))
        scratch += dbl(pltpu.SemaphoreType.DMA)
    scratch += (
        dbl(pltpu.VMEM((win,), jnp.int32))
        + dbl(pltpu.VMEM((nsc, 128), jnp.int32))
        + dbl(pltpu.VMEM((win,), jnp.float32))
        + dbl(pltpu.VMEM((win, table_w), table_dtype))
    )
    if need_sbuf:
        scratch += dbl(pltpu.VMEM((win, acc_w), jnp.float32))
    scratch += [
        pltpu.VMEM_SHARED((N_ACC, acc_w), acc_dtype),
        pltpu.SemaphoreType.DMA,
        pltpu.SemaphoreType.DMA,
        pltpu.SemaphoreType.DMA,
        pltpu.SemaphoreType.DMA,
    ]

    if phases == 1:
        out_type = jax.ShapeDtypeStruct((NCORE, N_ACC, acc_w), acc_dtype)
    else:
        out_type = jax.ShapeDtypeStruct((phases, NCORE, N_ACC, acc_w),
                                        acc_dtype)

    @functools.partial(pl.kernel, out_type=out_type, mesh=_sc_mesh(),
                       scratch_types=scratch,
                       compiler_params=pltpu.CompilerParams(
                           use_tc_tiling_on_sc=False,
                           needs_layout_passes=False))
    def body(*args):
        if use_pat:
            table_hbm, col_hbm, row3d_hbm, v_hbm, pat_hbm, out_hbm = args[:6]
            refs = list(args[6:])
            patbuf = [refs.pop(0), refs.pop(0)]
            psem = [refs.pop(0), refs.pop(0)]
        else:
            table_hbm, col_hbm, row3d_hbm, v_hbm, out_hbm = args[:5]
            refs = list(args[5:])
        colbuf = [refs.pop(0), refs.pop(0)]
        rowbuf2d = [refs.pop(0), refs.pop(0)]
        vbuf = [refs.pop(0), refs.pop(0)]
        gbuf = [refs.pop(0), refs.pop(0)]
        idxbuf = colbuf
        if need_sbuf:
            sbuf = [refs.pop(0), refs.pop(0)]
        else:
            sbuf = gbuf
        acc = refs.pop(0)
        gsem = [refs.pop(0), refs.pop(0)]
        ssem = [refs.pop(0), refs.pop(0)]

        c = lax.axis_index("c")
        s = lax.axis_index("s")
        base = s * PER_SUB
        zw = 32 if acc_dtype == jnp.bfloat16 else 16

        def zero_acc():
            @pl.loop(0, zchunk)
            def _z(i):
                for k in range(acc_w // zw):
                    sbuf[0][i, pl.ds(k * zw, zw)] = jnp.zeros((zw,), acc_dtype)

            for j in range(nz):
                r0 = pl.multiple_of(s * ROWS_PER_SUB + j * zchunk, 8)
                pltpu.sync_copy(sbuf[0].at[pl.ds(0, zchunk)],
                                acc.at[pl.ds(r0, zchunk)])
            plsc.subcore_barrier()

        def load_inputs(b, w):
            off = pl.multiple_of(base + w * win, win)
            widx = s * nwin + w
            pltpu.sync_copy(col_hbm.at[pl.ds(off, win)], colbuf[b])
            pltpu.sync_copy(row3d_hbm.at[widx], rowbuf2d[b])
            pltpu.sync_copy(v_hbm.at[pl.ds(off, win)], vbuf[b])
            if use_pat:
                return [pltpu.async_copy(pat_hbm.at[rowbuf2d[b].at[j]],
                                         patbuf[b].at[j], psem[b])
                        for j in range(nsc)]
            return None

        def compute_idx(b, hpat, qoff):
            coff = c * core_off + qoff

            @pl.loop(0, win // 16)
            def _i(k):
                col16 = colbuf[b][pl.ds(k * 16, 16)]
                idxbuf[b][pl.ds(k * 16, 16)] = col16 * idx_mult + coff

            if use_pat:
                for h in hpat:
                    h.wait()
                for j in range(nsc):
                    @pl.loop(0, 8)
                    def _p(k, j=j):
                        pat16 = patbuf[b][j, pl.ds(k * 16, 16)]
                        sl = pl.ds(j * 128 + k * 16, 16)
                        idxbuf[b][sl] = idxbuf[b][sl] + pat16

        def fire_gather(b):
            return pltpu.async_copy(table_hbm.at[idxbuf[b]], gbuf[b], gsem[b])

        def scale(b):
            @pl.loop(0, win // 16)
            def _r(k16):
                v16 = vbuf[b][pl.ds(k16 * 16, 16)]
                for lane in range(16):
                    i = k16 * 16 + lane
                    sv = v16[lane]
                    if table_dtype == jnp.float32:
                        for k in range(table_w // 16):
                            sl = pl.ds(k * 16, 16)
                            gbuf[b][i, sl] = gbuf[b][i, sl] * sv
                    else:
                        for k in range(table_w // 32):
                            sl = pl.ds(k * 32, 32)
                            a, b2 = plsc.unpack(
                                gbuf[b][i, sl],
                                format=plsc.PackFormat.INTERLEAVED)
                            a = a * sv
                            b2 = b2 * sv
                            if need_sbuf:
                                sbuf[b][i, pl.ds(k * 32, 16)] = a
                                sbuf[b][i, pl.ds(k * 32 + 16, 16)] = b2
                            else:
                                gbuf[b][i, sl] = plsc.pack(
                                    a, b2, format=plsc.PackFormat.INTERLEAVED)

        def fire_scatters(b):
            hs = []
            for j in range(nsc):
                hs.append(pltpu.async_copy(
                    sbuf[b].at[pl.ds(j * 128, 128)],
                    acc.at[rowbuf2d[b].at[j]], ssem[b], add=True))
            return hs

        for q in range(phases):
            zero_acc()
            qoff = q * (NCORE * core_off)

            @pl.loop(0, nwin // 2)
            def _w(wp, qoff=qoff):
                w0 = wp * 2
                hp0 = load_inputs(0, w0)
                hp1 = load_inputs(1, w0 + 1)
                compute_idx(0, hp0, qoff)
                hg0 = fire_gather(0)
                compute_idx(1, hp1, qoff)
                hg1 = fire_gather(1)
                hg0.wait()
                scale(0)
                hs0 = fire_scatters(0)
                hg1.wait()
                scale(1)
                hs1 = fire_scatters(1)
                for h in hs0 + hs1:
                    h.wait()

            plsc.subcore_barrier()
            dst = out_hbm.at[c] if phases == 1 else out_hbm.at[q].at[c]
            for j in range(nz):
                r0 = pl.multiple_of(s * ROWS_PER_SUB + j * zchunk, 8)
                pltpu.sync_copy(acc.at[pl.ds(r0, zchunk)],
                                dst.at[pl.ds(r0, zchunk)])

    return body


_PASS_A = _make_sc_pass(2 * N, 16, jnp.float32, jnp.float32,
                        core_off=N, idx_mult=1, use_pat=False, win=512,
                        phases=2)
_PASS_B = _make_sc_pass(2 * N, 64, jnp.bfloat16, jnp.bfloat16,
                        core_off=N, idx_mult=1, use_pat=False, win=256,
                        phases=2)
_PASS_C = _make_sc_pass(2 * 7 * N, 32, jnp.bfloat16, jnp.bfloat16,
                        core_off=7 * N, idx_mult=7, use_pat=True, win=512)


def _dense_body(emb_ref, side_ref, w_ref, b_ref, wg_ref, bg_ref, out_ref):
    x = emb_ref[...] + side_ref[...]
    t = jnp.dot(x, w_ref[...], preferred_element_type=jnp.float32)
    t = t + b_ref[...][None, :]
    t = jnp.where(t >= 0, t, 0.01 * t)
    sc = jnp.dot(t, wg_ref[...], preferred_element_type=jnp.float32)
    out_ref[...] = sc + bg_ref[...][None, :]


def _loss_body(u_ref, p_ref, n_ref, eu_ref, ep_ref, en_ref, bpr_ref, reg_ref):
    u = u_ref[...]
    p = p_ref[...]
    n = n_ref[...]
    pos = jnp.sum(u * p, axis=1)
    neg = jnp.sum(u * n, axis=1)
    x = neg - pos
    sp = jnp.maximum(x, 0.0) + jnp.log1p(jnp.exp(-jnp.abs(x)))
    bpr_ref[...] = jnp.mean(sp).reshape(1, 1)
    reg = 0.5 * (jnp.sum(eu_ref[...] ** 2) + jnp.sum(ep_ref[...] ** 2)
                 + jnp.sum(en_ref[...] ** 2)) / B
    reg_ref[...] = (REG_LAMBDA * reg).reshape(1, 1)


def kernel(user, positive, negative, edge_index, edge_values, user_table,
           item_table, fc_W, fc_b, fcg_W, fcg_b):
    row = edge_index[0].astype(jnp.int32)
    col = edge_index[1].astype(jnp.int32)
    v = edge_values

    npad = E_PAD - E
    padi = jnp.arange(npad, dtype=jnp.int32) % N
    rowp = jnp.concatenate([row, padi])
    colp = jnp.concatenate([col, padi])
    vp = jnp.concatenate([v, jnp.zeros((npad,), jnp.float32)])
    row3d_256 = rowp.reshape(E_PAD // 256, 2, 128)
    row3d_512 = rowp.reshape(E_PAD // 512, 4, 128)

    emb = jnp.concatenate([user_table, item_table], axis=0)

    emb_tbl = jnp.stack([emb[:, 0:16], emb[:, 16:32],
                         emb[:, 32:48], emb[:, 48:64]]).reshape(4 * N, 16)
    oa = _PASS_A(emb_tbl, colp, row3d_512, vp)
    side = jnp.concatenate([oa[0, 0, :N], oa[0, 1, :N],
                            oa[1, 0, :N], oa[1, 1, :N]], axis=1)

    wg8 = jnp.zeros((D, 8), jnp.float32).at[:, :G].set(fcg_W)
    bg8 = jnp.zeros((8,), jnp.float32).at[:G].set(fcg_b)
    scores8 = pl.pallas_call(
        _dense_body,
        grid=(25,),
        in_specs=[
            pl.BlockSpec((2000, D), lambda i: (i, 0)),
            pl.BlockSpec((2000, D), lambda i: (i, 0)),
            pl.BlockSpec((D, D), lambda i: (0, 0)),
            pl.BlockSpec((D,), lambda i: (0,)),
            pl.BlockSpec((D, 8), lambda i: (0, 0)),
            pl.BlockSpec((8,), lambda i: (0,)),
        ],
        out_specs=pl.BlockSpec((2000, 8), lambda i: (i, 0)),
        out_shape=jax.ShapeDtypeStruct((N, 8), jnp.float32),
    )(emb, side, fc_W, fc_b, wg8, bg8)

    scores = scores8[:, :G]
    top = jnp.max(scores, axis=1, keepdims=True)
    m = (scores == top).astype(jnp.float32)
    is_item = (jnp.arange(N) >= NUM_USERS)[:, None]
    m = jnp.where(is_item, 1.0, m)
    pat = (m[:, 0] + 2.0 * m[:, 1] + 4.0 * m[:, 2]).astype(jnp.int32) - 1

    mx = m[:, :, None] * emb[:, None, :]
    tq = []
    for q in range(2):
        for k in range(2):
            feats = mx[:, :, q * 32 + k * 16: q * 32 + (k + 1) * 16]
            tq.append(jnp.concatenate(
                [feats[:, 0], feats[:, 1], feats[:, 2],
                 jnp.zeros((N, 16), jnp.float32)], axis=1))
    table_b = jnp.stack(tq).reshape(4 * N, 64).astype(jnp.bfloat16)
    ob = _PASS_B(table_b, colp, row3d_256, vp)
    s_halves = [ob[q, :, :N].astype(jnp.float32) for q in range(2)]

    s_parts = []
    for q in range(2):
        oq = s_halves[q][:, :, :48].reshape(2, N, 3, 16)
        s_parts.append(jnp.concatenate([oq[0], oq[1]], axis=2))
    s = jnp.concatenate(s_parts, axis=2)

    cur = m[:, :, None] * s
    layer1 = cur.sum(axis=1)

    c0, c1, c2 = cur[:, 0], cur[:, 1], cur[:, 2]
    curp = jnp.stack([c0, c1, c0 + c1, c2, c0 + c2, c1 + c2, c0 + c1 + c2],
                     axis=1)
    curp2 = jnp.stack([curp[:, :, :32], curp[:, :, 32:]])
    curp_tbl = curp2.reshape(2 * 7 * N, 32).astype(jnp.bfloat16)
    l2 = _PASS_C(curp_tbl, colp, row3d_512, vp, pat)
    layer2 = jnp.concatenate([l2[0, :N], l2[1, :N]],
                             axis=1).astype(jnp.float32)

    final = (G * emb + layer1 + layer2) * (1.0 / 3.0)
    users_emb, items_emb = final[:NUM_USERS], final[NUM_USERS:]

    u = users_emb[user]
    p = items_emb[positive]
    n = items_emb[negative]
    ego_u = user_table[user]
    ego_p = item_table[positive]
    ego_n = item_table[negative]

    bpr, reg = pl.pallas_call(
        _loss_body,
        out_shape=[jax.ShapeDtypeStruct((1, 1), jnp.float32),
                   jax.ShapeDtypeStruct((1, 1), jnp.float32)],
    )(u, p, n, ego_u, ego_p, ego_n)
    return (bpr[0, 0], reg[0, 0])


# speedup vs baseline: 1.0417x; 1.0417x over previous
import functools

import jax
import jax.numpy as jnp
from jax import lax
from jax.experimental import pallas as pl
from jax.experimental.pallas import tpu as pltpu
from jax.experimental.pallas import tpu_sc as plsc

NUM_USERS = 25000
NUM_ITEMS = 25000
N = NUM_USERS + NUM_ITEMS
E = 800000
D = 64
G = 3
REG_LAMBDA = 1e-4
B = 4096

NSUB = 16
NCORE = 2
WIN = 512
PER_SUB = 50176
NWIN = PER_SUB // WIN
E_PAD = NSUB * PER_SUB
N_ACC = 50176
ROWS_PER_SUB = N_ACC // NSUB
ZCHUNK = 448


def _sc_mesh():
    return plsc.VectorSubcoreMesh(core_axis_name="c", subcore_axis_name="s")


def _make_sc_pass(table_rows, table_w, table_dtype, acc_dtype, core_off,
                  idx_mult, use_pat, win, phases=1):
    need_sbuf = table_dtype == jnp.bfloat16 and acc_dtype == jnp.float32
    acc_w = table_w
    nwin = PER_SUB // win
    nsc = win // 128
    zchunk = 448 if win >= 448 else (224 if win >= 224 else 112)
    nz = ROWS_PER_SUB // zchunk

    def dbl(t):
        return [t, t]

    scratch = []
    if use_pat:
        scratch += dbl(pltpu.VMEM((nsc, 128), jnp.int32))
        scratch += dbl(pltpu.SemaphoreType.DMA)
    scratch += (
        dbl(pltpu.VMEM((win,), jnp.int32))
        + dbl(pltpu.VMEM((nsc, 128), jnp.int32))
        + dbl(pltpu.VMEM((win,), jnp.float32))
        + dbl(pltpu.VMEM((win, table_w), table_dtype))
    )
    if need_sbuf:
        scratch += dbl(pltpu.VMEM((win, acc_w), jnp.float32))
    scratch += [
        pltpu.VMEM_SHARED((N_ACC, acc_w), acc_dtype),
        pltpu.SemaphoreType.DMA,
        pltpu.SemaphoreType.DMA,
        pltpu.SemaphoreType.DMA,
        pltpu.SemaphoreType.DMA,
    ]

    if phases == 1:
        out_type = jax.ShapeDtypeStruct((NCORE, N_ACC, acc_w), acc_dtype)
    else:
        out_type = jax.ShapeDtypeStruct((phases, NCORE, N_ACC, acc_w),
                                        acc_dtype)

    @functools.partial(pl.kernel, out_type=out_type, mesh=_sc_mesh(),
                       scratch_types=scratch,
                       compiler_params=pltpu.CompilerParams(
                           use_tc_tiling_on_sc=False,
                           needs_layout_passes=False))
    def body(*args):
        if use_pat:
            table_hbm, col_hbm, row3d_hbm, v_hbm, pat_hbm, out_hbm = args[:6]
            refs = list(args[6:])
            patbuf = [refs.pop(0), refs.pop(0)]
            psem = [refs.pop(0), refs.pop(0)]
        else:
            table_hbm, col_hbm, row3d_hbm, v_hbm, out_hbm = args[:5]
            refs = list(args[5:])
        colbuf = [refs.pop(0), refs.pop(0)]
        rowbuf2d = [refs.pop(0), refs.pop(0)]
        vbuf = [refs.pop(0), refs.pop(0)]
        gbuf = [refs.pop(0), refs.pop(0)]
        idxbuf = colbuf
        if need_sbuf:
            sbuf = [refs.pop(0), refs.pop(0)]
        else:
            sbuf = gbuf
        acc = refs.pop(0)
        gsem = [refs.pop(0), refs.pop(0)]
        ssem = [refs.pop(0), refs.pop(0)]

        c = lax.axis_index("c")
        s = lax.axis_index("s")
        base = s * PER_SUB
        zw = 32 if acc_dtype == jnp.bfloat16 else 16

        def zero_acc():
            @pl.loop(0, zchunk)
            def _z(i):
                for k in range(acc_w // zw):
                    sbuf[0][i, pl.ds(k * zw, zw)] = jnp.zeros((zw,), acc_dtype)

            for j in range(nz):
                r0 = pl.multiple_of(s * ROWS_PER_SUB + j * zchunk, 8)
                pltpu.sync_copy(sbuf[0].at[pl.ds(0, zchunk)],
                                acc.at[pl.ds(r0, zchunk)])
            plsc.subcore_barrier()

        def load_inputs(b, w):
            off = pl.multiple_of(base + w * win, win)
            widx = s * nwin + w
            pltpu.sync_copy(col_hbm.at[pl.ds(off, win)], colbuf[b])
            pltpu.sync_copy(row3d_hbm.at[widx], rowbuf2d[b])
            pltpu.sync_copy(v_hbm.at[pl.ds(off, win)], vbuf[b])
            if use_pat:
                return [pltpu.async_copy(pat_hbm.at[rowbuf2d[b].at[j]],
                                         patbuf[b].at[j], psem[b])
                        for j in range(nsc)]
            return None

        def compute_idx(b, hpat, qoff):
            coff = c * core_off + qoff

            @pl.loop(0, win // 16)
            def _i(k):
                col16 = colbuf[b][pl.ds(k * 16, 16)]
                idxbuf[b][pl.ds(k * 16, 16)] = col16 * idx_mult + coff

            if use_pat:
                for h in hpat:
                    h.wait()
                for j in range(nsc):
                    @pl.loop(0, 8)
                    def _p(k, j=j):
                        pat16 = patbuf[b][j, pl.ds(k * 16, 16)]
                        sl = pl.ds(j * 128 + k * 16, 16)
                        idxbuf[b][sl] = idxbuf[b][sl] + pat16

        def fire_gather(b):
            return pltpu.async_copy(table_hbm.at[idxbuf[b]], gbuf[b], gsem[b])

        def scale(b):
            @pl.loop(0, win // 16)
            def _r(k16):
                v16 = vbuf[b][pl.ds(k16 * 16, 16)]
                for lane in range(16):
                    i = k16 * 16 + lane
                    sv = v16[lane]
                    if table_dtype == jnp.float32:
                        for k in range(table_w // 16):
                            sl = pl.ds(k * 16, 16)
                            gbuf[b][i, sl] = gbuf[b][i, sl] * sv
                    else:
                        for k in range(table_w // 32):
                            sl = pl.ds(k * 32, 32)
                            a, b2 = plsc.unpack(
                                gbuf[b][i, sl],
                                format=plsc.PackFormat.INTERLEAVED)
                            a = a * sv
                            b2 = b2 * sv
                            if need_sbuf:
                                sbuf[b][i, pl.ds(k * 32, 16)] = a
                                sbuf[b][i, pl.ds(k * 32 + 16, 16)] = b2
                            else:
                                gbuf[b][i, sl] = plsc.pack(
                                    a, b2, format=plsc.PackFormat.INTERLEAVED)

        def fire_scatters(b):
            hs = []
            for j in range(nsc):
                hs.append(pltpu.async_copy(
                    sbuf[b].at[pl.ds(j * 128, 128)],
                    acc.at[rowbuf2d[b].at[j]], ssem[b], add=True))
            return hs

        for q in range(phases):
            zero_acc()
            qoff = q * (NCORE * core_off)

            @pl.loop(0, nwin // 2)
            def _w(wp, qoff=qoff):
                w0 = wp * 2
                hp0 = load_inputs(0, w0)
                hp1 = load_inputs(1, w0 + 1)
                compute_idx(0, hp0, qoff)
                hg0 = fire_gather(0)
                compute_idx(1, hp1, qoff)
                hg1 = fire_gather(1)
                hg0.wait()
                scale(0)
                hs0 = fire_scatters(0)
                hg1.wait()
                scale(1)
                hs1 = fire_scatters(1)
                for h in hs0 + hs1:
                    h.wait()

            plsc.subcore_barrier()
            dst = out_hbm.at[c] if phases == 1 else out_hbm.at[q].at[c]
            for j in range(nz):
                r0 = pl.multiple_of(s * ROWS_PER_SUB + j * zchunk, 8)
                pltpu.sync_copy(acc.at[pl.ds(r0, zchunk)],
                                dst.at[pl.ds(r0, zchunk)])

    return body


_PASS_A = _make_sc_pass(2 * N, 16, jnp.float32, jnp.float32,
                        core_off=N, idx_mult=1, use_pat=False, win=512)
_PASS_B = _make_sc_pass(2 * N, 64, jnp.bfloat16, jnp.bfloat16,
                        core_off=N, idx_mult=1, use_pat=False, win=256)
_PASS_C = _make_sc_pass(2 * 7 * N, 32, jnp.bfloat16, jnp.bfloat16,
                        core_off=7 * N, idx_mult=7, use_pat=True, win=512)


def _dense_body(emb_ref, side_ref, w_ref, b_ref, wg_ref, bg_ref, out_ref):
    x = emb_ref[...] + side_ref[...]
    t = jnp.dot(x, w_ref[...], preferred_element_type=jnp.float32)
    t = t + b_ref[...][None, :]
    t = jnp.where(t >= 0, t, 0.01 * t)
    sc = jnp.dot(t, wg_ref[...], preferred_element_type=jnp.float32)
    out_ref[...] = sc + bg_ref[...][None, :]


def _loss_body(u_ref, p_ref, n_ref, eu_ref, ep_ref, en_ref, bpr_ref, reg_ref):
    u = u_ref[...]
    p = p_ref[...]
    n = n_ref[...]
    pos = jnp.sum(u * p, axis=1)
    neg = jnp.sum(u * n, axis=1)
    x = neg - pos
    sp = jnp.maximum(x, 0.0) + jnp.log1p(jnp.exp(-jnp.abs(x)))
    bpr_ref[...] = jnp.mean(sp).reshape(1, 1)
    reg = 0.5 * (jnp.sum(eu_ref[...] ** 2) + jnp.sum(ep_ref[...] ** 2)
                 + jnp.sum(en_ref[...] ** 2)) / B
    reg_ref[...] = (REG_LAMBDA * reg).reshape(1, 1)


def kernel(user, positive, negative, edge_index, edge_values, user_table,
           item_table, fc_W, fc_b, fcg_W, fcg_b):
    row = edge_index[0].astype(jnp.int32)
    col = edge_index[1].astype(jnp.int32)
    v = edge_values

    npad = E_PAD - E
    padi = jnp.arange(npad, dtype=jnp.int32) % N
    rowp = jnp.concatenate([row, padi])
    colp = jnp.concatenate([col, padi])
    vp = jnp.concatenate([v, jnp.zeros((npad,), jnp.float32)])
    row3d_256 = rowp.reshape(E_PAD // 256, 2, 128)
    row3d_512 = rowp.reshape(E_PAD // 512, 4, 128)

    emb = jnp.concatenate([user_table, item_table], axis=0)

    side_parts = []
    for q in range(2):
        tq = jnp.stack([emb[:, q * 32: q * 32 + 16],
                        emb[:, q * 32 + 16: q * 32 + 32]]).reshape(2 * N, 16)
        oq = _PASS_A(tq, colp, row3d_512, vp)
        side_parts += [oq[0, :N], oq[1, :N]]
    side = jnp.concatenate(side_parts, axis=1)

    wg8 = jnp.zeros((D, 8), jnp.float32).at[:, :G].set(fcg_W)
    bg8 = jnp.zeros((8,), jnp.float32).at[:G].set(fcg_b)
    scores8 = pl.pallas_call(
        _dense_body,
        grid=(25,),
        in_specs=[
            pl.BlockSpec((2000, D), lambda i: (i, 0)),
            pl.BlockSpec((2000, D), lambda i: (i, 0)),
            pl.BlockSpec((D, D), lambda i: (0, 0)),
            pl.BlockSpec((D,), lambda i: (0,)),
            pl.BlockSpec((D, 8), lambda i: (0, 0)),
            pl.BlockSpec((8,), lambda i: (0,)),
        ],
        out_specs=pl.BlockSpec((2000, 8), lambda i: (i, 0)),
        out_shape=jax.ShapeDtypeStruct((N, 8), jnp.float32),
    )(emb, side, fc_W, fc_b, wg8, bg8)

    scores = scores8[:, :G]
    top = jnp.max(scores, axis=1, keepdims=True)
    m = (scores == top).astype(jnp.float32)
    is_item = (jnp.arange(N) >= NUM_USERS)[:, None]
    m = jnp.where(is_item, 1.0, m)
    pat = (m[:, 0] + 2.0 * m[:, 1] + 4.0 * m[:, 2]).astype(jnp.int32) - 1

    mx = m[:, :, None] * emb[:, None, :]
    s_halves = []
    for q in range(2):
        tq = []
        for k in range(2):
            feats = mx[:, :, q * 32 + k * 16: q * 32 + (k + 1) * 16]
            tq.append(jnp.concatenate(
                [feats[:, 0], feats[:, 1], feats[:, 2],
                 jnp.zeros((N, 16), jnp.float32)], axis=1))
        table_q = jnp.stack(tq).reshape(2 * N, 64).astype(jnp.bfloat16)
        out_q = _PASS_B(table_q, colp, row3d_256, vp)
        s_halves.append(out_q[:, :N].astype(jnp.float32))

    s_parts = []
    for q in range(2):
        oq = s_halves[q][:, :, :48].reshape(2, N, 3, 16)
        s_parts.append(jnp.concatenate([oq[0], oq[1]], axis=2))
    s = jnp.concatenate(s_parts, axis=2)

    cur = m[:, :, None] * s
    layer1 = cur.sum(axis=1)

    c0, c1, c2 = cur[:, 0], cur[:, 1], cur[:, 2]
    curp = jnp.stack([c0, c1, c0 + c1, c2, c0 + c2, c1 + c2, c0 + c1 + c2],
                     axis=1)
    curp2 = jnp.stack([curp[:, :, :32], curp[:, :, 32:]])
    curp_tbl = curp2.reshape(2 * 7 * N, 32).astype(jnp.bfloat16)
    l2 = _PASS_C(curp_tbl, colp, row3d_512, vp, pat)
    layer2 = jnp.concatenate([l2[0, :N], l2[1, :N]],
                             axis=1).astype(jnp.float32)

    final = (G * emb + layer1 + layer2) * (1.0 / 3.0)
    users_emb, items_emb = final[:NUM_USERS], final[NUM_USERS:]

    u = users_emb[user]
    p = items_emb[positive]
    n = items_emb[negative]
    ego_u = user_table[user]
    ego_p = item_table[positive]
    ego_n = item_table[negative]

    bpr, reg = pl.pallas_call(
        _loss_body,
        out_shape=[jax.ShapeDtypeStruct((1, 1), jnp.float32),
                   jax.ShapeDtypeStruct((1, 1), jnp.float32)],
    )(u, p, n, ego_u, ego_p, ego_n)
    return (bpr[0, 0], reg[0, 0])


# speedup vs baseline: 1.3267x; 1.2736x over previous
import functools

import jax
import jax.numpy as jnp
from jax import lax
from jax.experimental import pallas as pl
from jax.experimental.pallas import tpu as pltpu
from jax.experimental.pallas import tpu_sc as plsc

NUM_USERS = 25000
NUM_ITEMS = 25000
N = NUM_USERS + NUM_ITEMS
E = 800000
D = 64
G = 3
REG_LAMBDA = 1e-4
B = 4096

NSUB = 16
NCORE = 2
WIN = 512
PER_SUB = 50176
NWIN = PER_SUB // WIN
E_PAD = NSUB * PER_SUB
N_ACC = 50176
ROWS_PER_SUB = N_ACC // NSUB
ZCHUNK = 448


def _sc_mesh():
    return plsc.VectorSubcoreMesh(core_axis_name="c", subcore_axis_name="s")


def _make_sc_pass(table_rows, table_w, table_dtype, acc_dtype, core_off,
                  idx_mult, use_pat, win, phases=1):
    need_sbuf = table_dtype == jnp.bfloat16 and acc_dtype == jnp.float32
    acc_w = table_w
    nwin = PER_SUB // win
    nsc = win // 128
    zchunk = 448 if win >= 448 else (224 if win >= 224 else 112)
    nz = ROWS_PER_SUB // zchunk

    def dbl(t):
        return [t, t]

    scratch = []
    if use_pat:
        scratch += dbl(pltpu.VMEM((nsc, 128), jnp.int32))
        scratch += dbl(pltpu.SemaphoreType.DMA)
    scratch += (
        dbl(pltpu.VMEM((win,), jnp.int32))
        + dbl(pltpu.VMEM((nsc, 128), jnp.int32))
        + dbl(pltpu.VMEM((win,), jnp.float32))
        + dbl(pltpu.VMEM((win, table_w), table_dtype))
    )
    if need_sbuf:
        scratch += dbl(pltpu.VMEM((win, acc_w), jnp.float32))
    scratch += [
        pltpu.VMEM_SHARED((N_ACC, acc_w), acc_dtype),
        pltpu.SemaphoreType.DMA,
        pltpu.SemaphoreType.DMA,
        pltpu.SemaphoreType.DMA,
        pltpu.SemaphoreType.DMA,
        pltpu.SemaphoreType.DMA,
        pltpu.SemaphoreType.DMA,
    ]

    if phases == 1:
        out_type = jax.ShapeDtypeStruct((NCORE, N_ACC, acc_w), acc_dtype)
    else:
        out_type = jax.ShapeDtypeStruct((phases, NCORE, N_ACC, acc_w),
                                        acc_dtype)

    @functools.partial(pl.kernel, out_type=out_type, mesh=_sc_mesh(),
                       scratch_types=scratch,
                       compiler_params=pltpu.CompilerParams(
                           use_tc_tiling_on_sc=False,
                           needs_layout_passes=False))
    def body(*args):
        if use_pat:
            table_hbm, col_hbm, row3d_hbm, v_hbm, pat_hbm, out_hbm = args[:6]
            refs = list(args[6:])
            patbuf = [refs.pop(0), refs.pop(0)]
            psem = [refs.pop(0), refs.pop(0)]
        else:
            table_hbm, col_hbm, row3d_hbm, v_hbm, out_hbm = args[:5]
            refs = list(args[5:])
        colbuf = [refs.pop(0), refs.pop(0)]
        rowbuf2d = [refs.pop(0), refs.pop(0)]
        vbuf = [refs.pop(0), refs.pop(0)]
        gbuf = [refs.pop(0), refs.pop(0)]
        idxbuf = colbuf
        if need_sbuf:
            sbuf = [refs.pop(0), refs.pop(0)]
        else:
            sbuf = gbuf
        acc = refs.pop(0)
        gsem = [refs.pop(0), refs.pop(0)]
        ssem = [refs.pop(0), refs.pop(0)]
        lsem = [refs.pop(0), refs.pop(0)]

        c = lax.axis_index("c")
        s = lax.axis_index("s")
        base = s * PER_SUB
        zw = 32 if acc_dtype == jnp.bfloat16 else 16

        def zero_acc():
            @pl.loop(0, zchunk)
            def _z(i):
                for k in range(acc_w // zw):
                    sbuf[0][i, pl.ds(k * zw, zw)] = jnp.zeros((zw,), acc_dtype)

            for j in range(nz):
                r0 = pl.multiple_of(s * ROWS_PER_SUB + j * zchunk, 8)
                pltpu.sync_copy(sbuf[0].at[pl.ds(0, zchunk)],
                                acc.at[pl.ds(r0, zchunk)])
            plsc.subcore_barrier()

        def load_inputs(b, w):
            off = pl.multiple_of(base + w * win, win)
            widx = s * nwin + w
            return [
                pltpu.async_copy(col_hbm.at[pl.ds(off, win)], colbuf[b],
                                 lsem[b]),
                pltpu.async_copy(row3d_hbm.at[widx], rowbuf2d[b], lsem[b]),
                pltpu.async_copy(v_hbm.at[pl.ds(off, win)], vbuf[b], lsem[b]),
            ]

        def fire_pat(b):
            if use_pat:
                return [pltpu.async_copy(pat_hbm.at[rowbuf2d[b].at[j]],
                                         patbuf[b].at[j], psem[b])
                        for j in range(nsc)]
            return None

        def compute_idx(b, hpat, qoff):
            coff = c * core_off + qoff

            @pl.loop(0, win // 16)
            def _i(k):
                col16 = colbuf[b][pl.ds(k * 16, 16)]
                idxbuf[b][pl.ds(k * 16, 16)] = col16 * idx_mult + coff

            if use_pat:
                for h in hpat:
                    h.wait()
                for j in range(nsc):
                    @pl.loop(0, 8)
                    def _p(k, j=j):
                        pat16 = patbuf[b][j, pl.ds(k * 16, 16)]
                        sl = pl.ds(j * 128 + k * 16, 16)
                        idxbuf[b][sl] = idxbuf[b][sl] + pat16

        def fire_gather(b):
            return pltpu.async_copy(table_hbm.at[idxbuf[b]], gbuf[b], gsem[b])

        def scale(b):
            @pl.loop(0, win // 16)
            def _r(k16):
                v16 = vbuf[b][pl.ds(k16 * 16, 16)]
                for lane in range(16):
                    i = k16 * 16 + lane
                    sv = v16[lane]
                    if table_dtype == jnp.float32:
                        for k in range(table_w // 16):
                            sl = pl.ds(k * 16, 16)
                            gbuf[b][i, sl] = gbuf[b][i, sl] * sv
                    else:
                        for k in range(table_w // 32):
                            sl = pl.ds(k * 32, 32)
                            a, b2 = plsc.unpack(
                                gbuf[b][i, sl],
                                format=plsc.PackFormat.INTERLEAVED)
                            a = a * sv
                            b2 = b2 * sv
                            if need_sbuf:
                                sbuf[b][i, pl.ds(k * 32, 16)] = a
                                sbuf[b][i, pl.ds(k * 32 + 16, 16)] = b2
                            else:
                                gbuf[b][i, sl] = plsc.pack(
                                    a, b2, format=plsc.PackFormat.INTERLEAVED)

        def fire_scatters(b):
            hs = []
            for j in range(nsc):
                hs.append(pltpu.async_copy(
                    sbuf[b].at[pl.ds(j * 128, 128)],
                    acc.at[rowbuf2d[b].at[j]], ssem[b], add=True))
            return hs

        for q in range(phases):
            zero_acc()
            qoff = q * (NCORE * core_off)

            @pl.loop(0, nwin // 2)
            def _w(wp, qoff=qoff):
                w0 = wp * 2
                hl0 = load_inputs(0, w0)
                hl1 = load_inputs(1, w0 + 1)
                for h in hl0:
                    h.wait()
                hp0 = fire_pat(0)
                compute_idx(0, hp0, qoff)
                hg0 = fire_gather(0)
                for h in hl1:
                    h.wait()
                hp1 = fire_pat(1)
                compute_idx(1, hp1, qoff)
                hg1 = fire_gather(1)
                hg0.wait()
                scale(0)
                hs0 = fire_scatters(0)
                hg1.wait()
                scale(1)
                hs1 = fire_scatters(1)
                for h in hs0 + hs1:
                    h.wait()

            plsc.subcore_barrier()
            dst = out_hbm.at[c] if phases == 1 else out_hbm.at[q].at[c]
            for j in range(nz):
                r0 = pl.multiple_of(s * ROWS_PER_SUB + j * zchunk, 8)
                pltpu.sync_copy(acc.at[pl.ds(r0, zchunk)],
                                dst.at[pl.ds(r0, zchunk)])

    return body


_PASS_A = _make_sc_pass(2 * N, 16, jnp.float32, jnp.float32,
                        core_off=N, idx_mult=1, use_pat=False, win=512)
_PASS_B = _make_sc_pass(2 * N, 64, jnp.bfloat16, jnp.bfloat16,
                        core_off=N, idx_mult=1, use_pat=False, win=256)
_PASS_C = _make_sc_pass(2 * 7 * N, 32, jnp.bfloat16, jnp.bfloat16,
                        core_off=7 * N, idx_mult=7, use_pat=True, win=512)


def _dense_body(emb_ref, side_ref, w_ref, b_ref, wg_ref, bg_ref, out_ref):
    x = emb_ref[...] + side_ref[...]
    t = jnp.dot(x, w_ref[...], preferred_element_type=jnp.float32)
    t = t + b_ref[...][None, :]
    t = jnp.where(t >= 0, t, 0.01 * t)
    sc = jnp.dot(t, wg_ref[...], preferred_element_type=jnp.float32)
    out_ref[...] = sc + bg_ref[...][None, :]


def _loss_body(u_ref, p_ref, n_ref, eu_ref, ep_ref, en_ref, bpr_ref, reg_ref):
    u = u_ref[...]
    p = p_ref[...]
    n = n_ref[...]
    pos = jnp.sum(u * p, axis=1)
    neg = jnp.sum(u * n, axis=1)
    x = neg - pos
    sp = jnp.maximum(x, 0.0) + jnp.log1p(jnp.exp(-jnp.abs(x)))
    bpr_ref[...] = jnp.mean(sp).reshape(1, 1)
    reg = 0.5 * (jnp.sum(eu_ref[...] ** 2) + jnp.sum(ep_ref[...] ** 2)
                 + jnp.sum(en_ref[...] ** 2)) / B
    reg_ref[...] = (REG_LAMBDA * reg).reshape(1, 1)


def kernel(user, positive, negative, edge_index, edge_values, user_table,
           item_table, fc_W, fc_b, fcg_W, fcg_b):
    row = edge_index[0].astype(jnp.int32)
    col = edge_index[1].astype(jnp.int32)
    v = edge_values

    npad = E_PAD - E
    padi = jnp.arange(npad, dtype=jnp.int32) % N
    rowp = jnp.concatenate([row, padi])
    colp = jnp.concatenate([col, padi])
    vp = jnp.concatenate([v, jnp.zeros((npad,), jnp.float32)])
    row3d_256 = rowp.reshape(E_PAD // 256, 2, 128)
    row3d_512 = rowp.reshape(E_PAD // 512, 4, 128)

    emb = jnp.concatenate([user_table, item_table], axis=0)

    side_parts = []
    for q in range(2):
        tq = jnp.stack([emb[:, q * 32: q * 32 + 16],
                        emb[:, q * 32 + 16: q * 32 + 32]]).reshape(2 * N, 16)
        oq = _PASS_A(tq, colp, row3d_512, vp)
        side_parts += [oq[0, :N], oq[1, :N]]
    side = jnp.concatenate(side_parts, axis=1)

    wg8 = jnp.zeros((D, 8), jnp.float32).at[:, :G].set(fcg_W)
    bg8 = jnp.zeros((8,), jnp.float32).at[:G].set(fcg_b)
    scores8 = pl.pallas_call(
        _dense_body,
        grid=(25,),
        in_specs=[
            pl.BlockSpec((2000, D), lambda i: (i, 0)),
            pl.BlockSpec((2000, D), lambda i: (i, 0)),
            pl.BlockSpec((D, D), lambda i: (0, 0)),
            pl.BlockSpec((D,), lambda i: (0,)),
            pl.BlockSpec((D, 8), lambda i: (0, 0)),
            pl.BlockSpec((8,), lambda i: (0,)),
        ],
        out_specs=pl.BlockSpec((2000, 8), lambda i: (i, 0)),
        out_shape=jax.ShapeDtypeStruct((N, 8), jnp.float32),
    )(emb, side, fc_W, fc_b, wg8, bg8)

    scores = scores8[:, :G]
    top = jnp.max(scores, axis=1, keepdims=True)
    m = (scores == top).astype(jnp.float32)
    is_item = (jnp.arange(N) >= NUM_USERS)[:, None]
    m = jnp.where(is_item, 1.0, m)
    pat = (m[:, 0] + 2.0 * m[:, 1] + 4.0 * m[:, 2]).astype(jnp.int32) - 1

    mx = m[:, :, None] * emb[:, None, :]
    s_halves = []
    for q in range(2):
        tq = []
        for k in range(2):
            feats = mx[:, :, q * 32 + k * 16: q * 32 + (k + 1) * 16]
            tq.append(jnp.concatenate(
                [feats[:, 0], feats[:, 1], feats[:, 2],
                 jnp.zeros((N, 16), jnp.float32)], axis=1))
        table_q = jnp.stack(tq).reshape(2 * N, 64).astype(jnp.bfloat16)
        out_q = _PASS_B(table_q, colp, row3d_256, vp)
        s_halves.append(out_q[:, :N].astype(jnp.float32))

    s_parts = []
    for q in range(2):
        oq = s_halves[q][:, :, :48].reshape(2, N, 3, 16)
        s_parts.append(jnp.concatenate([oq[0], oq[1]], axis=2))
    s = jnp.concatenate(s_parts, axis=2)

    cur = m[:, :, None] * s
    layer1 = cur.sum(axis=1)

    c0, c1, c2 = cur[:, 0], cur[:, 1], cur[:, 2]
    curp = jnp.stack([c0, c1, c0 + c1, c2, c0 + c2, c1 + c2, c0 + c1 + c2],
                     axis=1)
    curp2 = jnp.stack([curp[:, :, :32], curp[:, :, 32:]])
    curp_tbl = curp2.reshape(2 * 7 * N, 32).astype(jnp.bfloat16)
    l2 = _PASS_C(curp_tbl, colp, row3d_512, vp, pat)
    layer2 = jnp.concatenate([l2[0, :N], l2[1, :N]],
                             axis=1).astype(jnp.float32)

    final = (G * emb + layer1 + layer2) * (1.0 / 3.0)
    users_emb, items_emb = final[:NUM_USERS], final[NUM_USERS:]

    u = users_emb[user]
    p = items_emb[positive]
    n = items_emb[negative]
    ego_u = user_table[user]
    ego_p = item_table[positive]
    ego_n = item_table[negative]

    bpr, reg = pl.pallas_call(
        _loss_body,
        out_shape=[jax.ShapeDtypeStruct((1, 1), jnp.float32),
                   jax.ShapeDtypeStruct((1, 1), jnp.float32)],
    )(u, p, n, ego_u, ego_p, ego_n)
    return (bpr[0, 0], reg[0, 0])


# speedup vs baseline: 1.3360x; 1.0069x over previous
import functools

import jax
import jax.numpy as jnp
from jax import lax
from jax.experimental import pallas as pl
from jax.experimental.pallas import tpu as pltpu
from jax.experimental.pallas import tpu_sc as plsc

NUM_USERS = 25000
NUM_ITEMS = 25000
N = NUM_USERS + NUM_ITEMS
E = 800000
D = 64
G = 3
REG_LAMBDA = 1e-4
B = 4096

NSUB = 16
NCORE = 2
WIN = 512
PER_SUB = 50176
NWIN = PER_SUB // WIN
E_PAD = NSUB * PER_SUB
N_ACC = 50176
ROWS_PER_SUB = N_ACC // NSUB
ZCHUNK = 448


def _sc_mesh():
    return plsc.VectorSubcoreMesh(core_axis_name="c", subcore_axis_name="s")


def _make_sc_pass(table_rows, table_w, table_dtype, acc_dtype, core_off,
                  idx_mult, use_pat, win, phases=1):
    need_sbuf = table_dtype == jnp.bfloat16 and acc_dtype == jnp.float32
    acc_w = table_w
    nwin = PER_SUB // win
    nsc = win // 128
    zchunk = 448 if win >= 448 else (224 if win >= 224 else 112)
    nz = ROWS_PER_SUB // zchunk

    def dbl(t):
        return [t, t]

    scratch = []
    if use_pat:
        scratch += dbl(pltpu.VMEM((nsc, 128), jnp.int32))
        scratch += dbl(pltpu.SemaphoreType.DMA)
    scratch += (
        dbl(pltpu.VMEM((win,), jnp.int32))
        + dbl(pltpu.VMEM((nsc, 128), jnp.int32))
        + dbl(pltpu.VMEM((win,), jnp.float32))
        + dbl(pltpu.VMEM((win, table_w), table_dtype))
    )
    if need_sbuf:
        scratch += dbl(pltpu.VMEM((win, acc_w), jnp.float32))
    scratch += [
        pltpu.VMEM_SHARED((N_ACC, acc_w), acc_dtype),
        pltpu.SemaphoreType.DMA,
        pltpu.SemaphoreType.DMA,
        pltpu.SemaphoreType.DMA,
        pltpu.SemaphoreType.DMA,
        pltpu.SemaphoreType.DMA,
        pltpu.SemaphoreType.DMA,
    ]

    if phases == 1:
        out_type = jax.ShapeDtypeStruct((NCORE, N_ACC, acc_w), acc_dtype)
    else:
        out_type = jax.ShapeDtypeStruct((phases, NCORE, N_ACC, acc_w),
                                        acc_dtype)

    @functools.partial(pl.kernel, out_type=out_type, mesh=_sc_mesh(),
                       scratch_types=scratch,
                       compiler_params=pltpu.CompilerParams(
                           use_tc_tiling_on_sc=False,
                           needs_layout_passes=False))
    def body(*args):
        if use_pat:
            table_hbm, col_hbm, row3d_hbm, v_hbm, pat_hbm, out_hbm = args[:6]
            refs = list(args[6:])
            patbuf = [refs.pop(0), refs.pop(0)]
            psem = [refs.pop(0), refs.pop(0)]
        else:
            table_hbm, col_hbm, row3d_hbm, v_hbm, out_hbm = args[:5]
            refs = list(args[5:])
        colbuf = [refs.pop(0), refs.pop(0)]
        rowbuf2d = [refs.pop(0), refs.pop(0)]
        vbuf = [refs.pop(0), refs.pop(0)]
        gbuf = [refs.pop(0), refs.pop(0)]
        idxbuf = colbuf
        if need_sbuf:
            sbuf = [refs.pop(0), refs.pop(0)]
        else:
            sbuf = gbuf
        acc = refs.pop(0)
        gsem = [refs.pop(0), refs.pop(0)]
        ssem = [refs.pop(0), refs.pop(0)]
        lsem = [refs.pop(0), refs.pop(0)]

        c = lax.axis_index("c")
        s = lax.axis_index("s")
        base = s * PER_SUB
        zw = 32 if acc_dtype == jnp.bfloat16 else 16

        def zero_acc():
            @pl.loop(0, zchunk)
            def _z(i):
                for k in range(acc_w // zw):
                    sbuf[0][i, pl.ds(k * zw, zw)] = jnp.zeros((zw,), acc_dtype)

            for j in range(nz):
                r0 = pl.multiple_of(s * ROWS_PER_SUB + j * zchunk, 8)
                pltpu.sync_copy(sbuf[0].at[pl.ds(0, zchunk)],
                                acc.at[pl.ds(r0, zchunk)])
            plsc.subcore_barrier()

        def load_inputs(b, w):
            off = pl.multiple_of(base + w * win, win)
            widx = s * nwin + w
            return [
                pltpu.async_copy(col_hbm.at[pl.ds(off, win)], colbuf[b],
                                 lsem[b]),
                pltpu.async_copy(row3d_hbm.at[widx], rowbuf2d[b], lsem[b]),
                pltpu.async_copy(v_hbm.at[pl.ds(off, win)], vbuf[b], lsem[b]),
            ]

        def fire_pat(b):
            if use_pat:
                return [pltpu.async_copy(pat_hbm.at[rowbuf2d[b].at[j]],
                                         patbuf[b].at[j], psem[b])
                        for j in range(nsc)]
            return None

        def compute_idx(b, hpat, qoff):
            coff = c * core_off + qoff

            @pl.loop(0, win // 16)
            def _i(k):
                col16 = colbuf[b][pl.ds(k * 16, 16)]
                idxbuf[b][pl.ds(k * 16, 16)] = col16 * idx_mult + coff

            if use_pat:
                for h in hpat:
                    h.wait()
                for j in range(nsc):
                    @pl.loop(0, 8)
                    def _p(k, j=j):
                        pat16 = patbuf[b][j, pl.ds(k * 16, 16)]
                        sl = pl.ds(j * 128 + k * 16, 16)
                        idxbuf[b][sl] = idxbuf[b][sl] + pat16

        def fire_gather(b):
            return pltpu.async_copy(table_hbm.at[idxbuf[b]], gbuf[b], gsem[b])

        def scale(b):
            @pl.loop(0, win // 16)
            def _r(k16):
                v16 = vbuf[b][pl.ds(k16 * 16, 16)]
                for lane in range(16):
                    i = k16 * 16 + lane
                    sv = v16[lane]
                    if table_dtype == jnp.float32:
                        for k in range(table_w // 16):
                            sl = pl.ds(k * 16, 16)
                            gbuf[b][i, sl] = gbuf[b][i, sl] * sv
                    else:
                        for k in range(table_w // 32):
                            sl = pl.ds(k * 32, 32)
                            a, b2 = plsc.unpack(
                                gbuf[b][i, sl],
                                format=plsc.PackFormat.INTERLEAVED)
                            a = a * sv
                            b2 = b2 * sv
                            if need_sbuf:
                                sbuf[b][i, pl.ds(k * 32, 16)] = a
                                sbuf[b][i, pl.ds(k * 32 + 16, 16)] = b2
                            else:
                                gbuf[b][i, sl] = plsc.pack(
                                    a, b2, format=plsc.PackFormat.INTERLEAVED)

        def fire_scatters(b):
            hs = []
            for j in range(nsc):
                hs.append(pltpu.async_copy(
                    sbuf[b].at[pl.ds(j * 128, 128)],
                    acc.at[rowbuf2d[b].at[j]], ssem[b], add=True))
            return hs

        for q in range(phases):
            zero_acc()
            qoff = q * (NCORE * core_off)

            @pl.loop(0, nwin // 2)
            def _w(wp, qoff=qoff):
                w0 = wp * 2
                hl0 = load_inputs(0, w0)
                hl1 = load_inputs(1, w0 + 1)
                for h in hl0:
                    h.wait()
                hp0 = fire_pat(0)
                compute_idx(0, hp0, qoff)
                hg0 = fire_gather(0)
                for h in hl1:
                    h.wait()
                hp1 = fire_pat(1)
                compute_idx(1, hp1, qoff)
                hg1 = fire_gather(1)
                hg0.wait()
                scale(0)
                hs0 = fire_scatters(0)
                hg1.wait()
                scale(1)
                hs1 = fire_scatters(1)
                for h in hs0 + hs1:
                    h.wait()

            plsc.subcore_barrier()
            dst = out_hbm.at[c] if phases == 1 else out_hbm.at[q].at[c]
            for j in range(nz):
                r0 = pl.multiple_of(s * ROWS_PER_SUB + j * zchunk, 8)
                pltpu.sync_copy(acc.at[pl.ds(r0, zchunk)],
                                dst.at[pl.ds(r0, zchunk)])

    return body


_PASS_A = _make_sc_pass(2 * N, 16, jnp.float32, jnp.float32,
                        core_off=N, idx_mult=1, use_pat=False, win=512)
_PASS_B = _make_sc_pass(2 * N, 64, jnp.bfloat16, jnp.bfloat16,
                        core_off=N, idx_mult=1, use_pat=False, win=256)
_PASS_C = _make_sc_pass(2 * 7 * N, 32, jnp.bfloat16, jnp.bfloat16,
                        core_off=7 * N, idx_mult=7, use_pat=True, win=512)


def _dense_body(emb_ref, side_ref, w_ref, b_ref, wg_ref, bg_ref, out_ref):
    x = emb_ref[...] + side_ref[...]
    t = jnp.dot(x, w_ref[...], preferred_element_type=jnp.float32)
    t = t + b_ref[...][None, :]
    t = jnp.where(t >= 0, t, 0.01 * t)
    sc = jnp.dot(t, wg_ref[...], preferred_element_type=jnp.float32)
    out_ref[...] = sc + bg_ref[...][None, :]


def _loss_body(u_ref, p_ref, n_ref, eu_ref, ep_ref, en_ref, bpr_ref, reg_ref):
    u = u_ref[...]
    p = p_ref[...]
    n = n_ref[...]
    pos = jnp.sum(u * p, axis=1)
    neg = jnp.sum(u * n, axis=1)
    x = neg - pos
    sp = jnp.maximum(x, 0.0) + jnp.log1p(jnp.exp(-jnp.abs(x)))
    bpr_ref[...] = jnp.mean(sp).reshape(1, 1)
    reg = 0.5 * (jnp.sum(eu_ref[...] ** 2) + jnp.sum(ep_ref[...] ** 2)
                 + jnp.sum(en_ref[...] ** 2)) / B
    reg_ref[...] = (REG_LAMBDA * reg).reshape(1, 1)


def kernel(user, positive, negative, edge_index, edge_values, user_table,
           item_table, fc_W, fc_b, fcg_W, fcg_b):
    row = edge_index[0].astype(jnp.int32)
    col = edge_index[1].astype(jnp.int32)
    v = edge_values

    npad = E_PAD - E
    padi = jnp.arange(npad, dtype=jnp.int32) % N
    rowp = jnp.concatenate([row, padi])
    colp = jnp.concatenate([col, padi])
    vp = jnp.concatenate([v, jnp.zeros((npad,), jnp.float32)])
    row3d_256 = rowp.reshape(E_PAD // 256, 2, 128)
    row3d_512 = rowp.reshape(E_PAD // 512, 4, 128)

    emb = jnp.concatenate([user_table, item_table], axis=0)

    side_parts = []
    for q in range(2):
        tq = jnp.stack([emb[:, q * 32: q * 32 + 16],
                        emb[:, q * 32 + 16: q * 32 + 32]]).reshape(2 * N, 16)
        oq = _PASS_A(tq, colp, row3d_512, vp)
        side_parts += [oq[0, :N], oq[1, :N]]
    side = jnp.concatenate(side_parts, axis=1)

    wg8 = jnp.zeros((D, 8), jnp.float32).at[:, :G].set(fcg_W)
    bg8 = jnp.zeros((8,), jnp.float32).at[:G].set(fcg_b)
    scores8 = pl.pallas_call(
        _dense_body,
        grid=(25,),
        in_specs=[
            pl.BlockSpec((2000, D), lambda i: (i, 0)),
            pl.BlockSpec((2000, D), lambda i: (i, 0)),
            pl.BlockSpec((D, D), lambda i: (0, 0)),
            pl.BlockSpec((D,), lambda i: (0,)),
            pl.BlockSpec((D, 8), lambda i: (0, 0)),
            pl.BlockSpec((8,), lambda i: (0,)),
        ],
        out_specs=pl.BlockSpec((2000, 8), lambda i: (i, 0)),
        out_shape=jax.ShapeDtypeStruct((N, 8), jnp.float32),
    )(emb, side, fc_W, fc_b, wg8, bg8)

    scores = scores8[:, :G]
    top = jnp.max(scores, axis=1, keepdims=True)
    m = (scores == top).astype(jnp.float32)
    is_item = (jnp.arange(N) >= NUM_USERS)[:, None]
    m = jnp.where(is_item, 1.0, m)
    pat = (m[:, 0] + 2.0 * m[:, 1] + 4.0 * m[:, 2]).astype(jnp.int32) - 1

    mx = m[:, :, None] * emb[:, None, :]
    s_halves = []
    for q in range(2):
        tq = []
        for k in range(2):
            feats = mx[:, :, q * 32 + k * 16: q * 32 + (k + 1) * 16]
            tq.append(jnp.concatenate(
                [feats[:, 0], feats[:, 1], feats[:, 2],
                 jnp.zeros((N, 16), jnp.float32)], axis=1))
        table_q = jnp.stack(tq).reshape(2 * N, 64).astype(jnp.bfloat16)
        out_q = _PASS_B(table_q, colp, row3d_256, vp)
        s_halves.append(out_q[:, :N].astype(jnp.float32))

    s_parts = []
    for q in range(2):
        oq = s_halves[q][:, :, :48].reshape(2, N, 3, 16)
        s_parts.append(jnp.concatenate([oq[0], oq[1]], axis=2))
    s = jnp.concatenate(s_parts, axis=2)

    cur = m[:, :, None] * s
    layer1 = cur.sum(axis=1)

    cb = cur.astype(jnp.bfloat16)
    c0, c1, c2 = cb[:, 0], cb[:, 1], cb[:, 2]
    curp = jnp.stack([c0, c1, c0 + c1, c2, c0 + c2, c1 + c2, c0 + c1 + c2],
                     axis=1)
    curp2 = jnp.stack([curp[:, :, :32], curp[:, :, 32:]])
    curp_tbl = curp2.reshape(2 * 7 * N, 32)
    l2 = _PASS_C(curp_tbl, colp, row3d_512, vp, pat)
    layer2 = jnp.concatenate([l2[0, :N], l2[1, :N]],
                             axis=1).astype(jnp.float32)

    final = (G * emb + layer1 + layer2) * (1.0 / 3.0)
    users_emb, items_emb = final[:NUM_USERS], final[NUM_USERS:]

    u = users_emb[user]
    p = items_emb[positive]
    n = items_emb[negative]
    ego_u = user_table[user]
    ego_p = item_table[positive]
    ego_n = item_table[negative]

    bpr, reg = pl.pallas_call(
        _loss_body,
        out_shape=[jax.ShapeDtypeStruct((1, 1), jnp.float32),
                   jax.ShapeDtypeStruct((1, 1), jnp.float32)],
    )(u, p, n, ego_u, ego_p, ego_n)
    return (bpr[0, 0], reg[0, 0])
